# Initial kernel scaffold; baseline (speedup 1.0000x reference)
import jax
import jax.numpy as jnp
from jax import lax
from jax.experimental import pallas as pl
from jax.experimental.pallas import tpu as pltpu
from jax.experimental.pallas import tpu_sc as plsc

NNODE = 10000
NEDGE = 320000
NPAD = 10240
CH = 128
NCH = NEDGE // CH
NC = 2
NS = 16
SLICE = NPAD // NS
DEG_BUF = 157 * CH
AGG_BUF = 79 * CH

f32 = jnp.float32
i32 = jnp.int32


def _rsqrt16(x):
    xi = plsc.bitcast(x, i32)
    yi = jnp.full((16,), 0x5F3759DF, i32) - lax.shift_right_logical(
        xi, jnp.ones((16,), i32))
    y = plsc.bitcast(yi, f32)
    for _ in range(3):
        y = y * (1.5 - 0.5 * x * y * y)
    return y


def _fill16(v):
    return jnp.full((16,), v, i32)


def _l1_body(src_hbm, dst_hbm, w_hbm, x_hbm, agg_hbm, dinv_hbm,
             dst_all, w_all, src_all, deg_acc, dinv_priv, red, rows,
             idx_s, idx_d, nrm, stage_sh, dinv_sh, acc_sh, sem):
    c = lax.axis_index("c")
    s = lax.axis_index("s")
    z16 = jnp.zeros((16,), f32)

    ks0 = (s * NCH) // NS
    ks1 = ((s + 1) * NCH) // NS

    @pl.loop(0, NPAD // 16)
    def _(j):
        deg_acc[pl.ds(j * 16, 16)] = z16

    pltpu.sync_copy(dst_hbm.at[pl.ds(ks0 * CH, DEG_BUF)], dst_all)
    pltpu.sync_copy(w_hbm.at[pl.ds(ks0 * CH, DEG_BUF)], w_all)

    @pl.loop(0, (ks1 - ks0) * (CH // 16))
    def _(g):
        d16 = dst_all[pl.ds(g * 16, 16)]
        w16 = w_all[pl.ds(g * 16, 16)]
        plsc.addupdate_scatter(deg_acc, [d16], w16)

    pltpu.sync_copy(deg_acc, stage_sh.at[s])

    @pl.loop(0, CH)
    def _(r):
        for j in range(8):
            rows[r, pl.ds(j * 16, 16)] = z16
    for m in range(SLICE // CH):
        pltpu.sync_copy(rows, acc_sh.at[pl.ds(s * SLICE + m * CH, CH), :])

    plsc.subcore_barrier()

    for r in range(NS):
        pltpu.sync_copy(stage_sh.at[r, pl.ds(s * SLICE, SLICE)], red.at[r])

    @pl.loop(0, SLICE // 16)
    def _(j):
        acc = jnp.full((16,), 1.0, f32)
        for r in range(NS):
            acc = acc + red[r, pl.ds(j * 16, 16)]
        dinv_priv[pl.ds(s * SLICE + j * 16, 16)] = _rsqrt16(acc)

    pltpu.sync_copy(dinv_priv.at[pl.ds(s * SLICE, SLICE)],
                    dinv_sh.at[pl.ds(s * SLICE, SLICE)])

    @pl.when(c == 0)
    def _():
        pltpu.sync_copy(dinv_priv.at[pl.ds(s * SLICE, SLICE)],
                        dinv_hbm.at[pl.ds(s * SLICE, SLICE)])

    plsc.subcore_barrier()
    pltpu.sync_copy(dinv_sh, dinv_priv)

    ka0 = c * (NCH // NC) + (s * (NCH // NC)) // NS
    ka1 = c * (NCH // NC) + ((s + 1) * (NCH // NC)) // NS
    pltpu.sync_copy(src_hbm.at[pl.ds(ka0 * CH, AGG_BUF)], src_all)
    pltpu.sync_copy(dst_hbm.at[pl.ds(ka0 * CH, AGG_BUF)],
                    dst_all.at[pl.ds(0, AGG_BUF)])
    pltpu.sync_copy(w_hbm.at[pl.ds(ka0 * CH, AGG_BUF)],
                    w_all.at[pl.ds(0, AGG_BUF)])

    @pl.loop(0, ka1 - ka0)
    def _(k):
        base = k * CH
        for g in range(8):
            idx_s[pl.ds(g * 16, 16)] = src_all[pl.ds(base + g * 16, 16)]
            idx_d[pl.ds(g * 16, 16)] = dst_all[pl.ds(base + g * 16, 16)]
        pltpu.async_copy(x_hbm.at[idx_s], rows, sem).wait()
        for g in range(8):
            s16 = idx_s[pl.ds(g * 16, 16)]
            d16 = idx_d[pl.ds(g * 16, 16)]
            w16 = w_all[pl.ds(base + g * 16, 16)]
            dvs = plsc.load_gather(dinv_priv, [s16])
            dvd = plsc.load_gather(dinv_priv, [d16])
            nrm[pl.ds(g * 16, 16)] = dvs * w16 * dvd

        @pl.loop(0, CH)
        def _(r):
            sp = plsc.load_gather(nrm, [_fill16(r)])
            for j in range(8):
                rows[r, pl.ds(j * 16, 16)] = rows[r, pl.ds(j * 16, 16)] * sp

        pltpu.sync_copy(rows, acc_sh.at[idx_d], add=True)

    plsc.subcore_barrier()
    pltpu.sync_copy(acc_sh.at[pl.ds(s * SLICE, SLICE), :],
                    agg_hbm.at[c, pl.ds(s * SLICE, SLICE), :])


def _sc_layer1(src, dst, w, x):
    mesh = plsc.VectorSubcoreMesh(core_axis_name="c", subcore_axis_name="s",
                                  num_cores=NC, num_subcores=NS)
    return pl.kernel(
        _l1_body,
        out_type=(jax.ShapeDtypeStruct((NC, NPAD, 128), f32),
                  jax.ShapeDtypeStruct((NPAD,), f32)),
        mesh=mesh,
        scratch_types=[
            pltpu.VMEM((DEG_BUF,), i32),
            pltpu.VMEM((DEG_BUF,), f32),
            pltpu.VMEM((AGG_BUF,), i32),
            pltpu.VMEM((NPAD,), f32),
            pltpu.VMEM((NPAD,), f32),
            pltpu.VMEM((NS, SLICE), f32),
            pltpu.VMEM((CH, 128), f32),
            pltpu.VMEM((CH,), i32),
            pltpu.VMEM((CH,), i32),
            pltpu.VMEM((CH,), f32),
            pltpu.VMEM_SHARED((NS, NPAD), f32),
            pltpu.VMEM_SHARED((NPAD,), f32),
            pltpu.VMEM_SHARED((NPAD, 128), f32),
            pltpu.SemaphoreType.DMA,
        ],
        name="gcn_sc_layer1",
    )(src, dst, w, x)


def _l2_body(src_hbm, dst_hbm, w_hbm, hws_hbm, agg_hbm,
             src_all, dst_all, w_all, rows, idx_s, idx_d, acc_sh, sem):
    c = lax.axis_index("c")
    s = lax.axis_index("s")
    z16 = jnp.zeros((16,), f32)

    @pl.loop(0, CH)
    def _(r):
        rows[r, pl.ds(0, 16)] = z16
        rows[r, pl.ds(16, 16)] = z16
    for m in range(SLICE // CH):
        pltpu.sync_copy(rows, acc_sh.at[pl.ds(s * SLICE + m * CH, CH), :])
    plsc.subcore_barrier()

    ka0 = c * (NCH // NC) + (s * (NCH // NC)) // NS
    ka1 = c * (NCH // NC) + ((s + 1) * (NCH // NC)) // NS
    pltpu.sync_copy(src_hbm.at[pl.ds(ka0 * CH, AGG_BUF)], src_all)
    pltpu.sync_copy(dst_hbm.at[pl.ds(ka0 * CH, AGG_BUF)], dst_all)
    pltpu.sync_copy(w_hbm.at[pl.ds(ka0 * CH, AGG_BUF)], w_all)

    @pl.loop(0, ka1 - ka0)
    def _(k):
        base = k * CH
        for g in range(8):
            idx_s[pl.ds(g * 16, 16)] = src_all[pl.ds(base + g * 16, 16)]
            idx_d[pl.ds(g * 16, 16)] = dst_all[pl.ds(base + g * 16, 16)]
        pltpu.async_copy(hws_hbm.at[idx_s], rows, sem).wait()

        @pl.loop(0, CH)
        def _(r):
            sp = plsc.load_gather(w_all, [_fill16(base + r)])
            rows[r, pl.ds(0, 16)] = rows[r, pl.ds(0, 16)] * sp
            rows[r, pl.ds(16, 16)] = rows[r, pl.ds(16, 16)] * sp

        pltpu.sync_copy(rows, acc_sh.at[idx_d], add=True)

    plsc.subcore_barrier()
    pltpu.sync_copy(acc_sh.at[pl.ds(s * SLICE, SLICE), :],
                    agg_hbm.at[c, pl.ds(s * SLICE, SLICE), :])


def _sc_layer2(src, dst, w, hws):
    mesh = plsc.VectorSubcoreMesh(core_axis_name="c", subcore_axis_name="s",
                                  num_cores=NC, num_subcores=NS)
    return pl.kernel(
        _l2_body,
        out_type=jax.ShapeDtypeStruct((NC, NPAD, 32), f32),
        mesh=mesh,
        scratch_types=[
            pltpu.VMEM((AGG_BUF,), i32),
            pltpu.VMEM((AGG_BUF,), i32),
            pltpu.VMEM((AGG_BUF,), f32),
            pltpu.VMEM((CH, 32), f32),
            pltpu.VMEM((CH,), i32),
            pltpu.VMEM((CH,), i32),
            pltpu.VMEM_SHARED((NPAD, 32), f32),
            pltpu.SemaphoreType.DMA,
        ],
        name="gcn_sc_layer2",
    )(src, dst, w, hws)


BM = 200


def _tc_mid_body(a0, a1, xr, dv, w1, b1, w2, hw_out, hws_out):
    d = dv[...]
    z = (a0[...] + a1[...]) * d + xr[...] * (d * d)
    h = jnp.dot(z, w1[...], preferred_element_type=f32) + b1[...]
    h = jnp.maximum(h, 0.0)
    hw = jnp.dot(h, w2[...], preferred_element_type=f32)
    hw_out[...] = hw
    hws_out[...] = hw * d


def _tc_mid(a0, a1, x, dinv_col, W1p, b1p, W2p):
    return pl.pallas_call(
        _tc_mid_body,
        grid=(NNODE // BM,),
        in_specs=[
            pl.BlockSpec((BM, 128), lambda i: (i, 0)),
            pl.BlockSpec((BM, 128), lambda i: (i, 0)),
            pl.BlockSpec((BM, 128), lambda i: (i, 0)),
            pl.BlockSpec((BM, 1), lambda i: (i, 0)),
            pl.BlockSpec((128, 256), lambda i: (0, 0)),
            pl.BlockSpec((1, 256), lambda i: (0, 0)),
            pl.BlockSpec((256, 32), lambda i: (0, 0)),
        ],
        out_specs=[
            pl.BlockSpec((BM, 32), lambda i: (i, 0)),
            pl.BlockSpec((BM, 32), lambda i: (i, 0)),
        ],
        out_shape=[
            jax.ShapeDtypeStruct((NNODE, 32), f32),
            jax.ShapeDtypeStruct((NNODE, 32), f32),
        ],
        name="gcn_tc_mid",
    )(a0, a1, x, dinv_col, W1p, b1p, W2p)


def _tc_fin_body(q0, q1, hwr, dv, b2, out):
    d = dv[...]
    out[...] = (q0[...] + q1[...]) * d + hwr[...] * (d * d) + b2[...]


def _tc_fin(q0, q1, hw, dinv_col, b2p):
    return pl.pallas_call(
        _tc_fin_body,
        grid=(NNODE // BM,),
        in_specs=[
            pl.BlockSpec((BM, 32), lambda i: (i, 0)),
            pl.BlockSpec((BM, 32), lambda i: (i, 0)),
            pl.BlockSpec((BM, 32), lambda i: (i, 0)),
            pl.BlockSpec((BM, 1), lambda i: (i, 0)),
            pl.BlockSpec((1, 32), lambda i: (0, 0)),
        ],
        out_specs=pl.BlockSpec((BM, 32), lambda i: (i, 0)),
        out_shape=jax.ShapeDtypeStruct((NNODE, 32), f32),
        name="gcn_tc_fin",
    )(q0, q1, hw, dinv_col, b2p)


def kernel(x, edge_index, edge_weight, W1, b1, W2, b2):
    src = edge_index[0]
    dst = edge_index[1]
    agg1, dinv = _sc_layer1(src, dst, edge_weight, x)
    dinv_col = dinv[:NNODE].reshape(NNODE, 1)
    W1p = jnp.pad(W1, ((0, 0), (0, 56)))
    b1p = jnp.pad(b1, (0, 56)).reshape(1, 256)
    W2p = jnp.pad(W2, ((0, 56), (0, 12)))
    hw, hws = _tc_mid(agg1[0, :NNODE], agg1[1, :NNODE], x, dinv_col,
                      W1p, b1p, W2p)
    agg2 = _sc_layer2(src, dst, edge_weight, hws)
    b2p = jnp.pad(b2, (0, 12)).reshape(1, 32)
    out = _tc_fin(agg2[0, :NNODE], agg2[1, :NNODE], hw, dinv_col, b2p)
    return out[:, :20]


# speedup vs baseline: 18.6620x; 18.6620x over previous
import jax
import jax.numpy as jnp
from jax import lax
from jax.experimental import pallas as pl
from jax.experimental.pallas import tpu as pltpu
from jax.experimental.pallas import tpu_sc as plsc

NNODE = 10000
NEDGE = 320000
NPAD = 10240
CH = 128
NCH = NEDGE // CH
NC = 2
NS = 16
SLICE = NPAD // NS
SEG = 40
SEGE = SEG * CH
EBUF = (NCH + SEG) * CH
AGG_BUF = 79 * CH

f32 = jnp.float32
i32 = jnp.int32


def _rsqrt16(x):
    xi = plsc.bitcast(x, i32)
    yi = jnp.full((16,), 0x5F3759DF, i32) - lax.shift_right_logical(
        xi, jnp.ones((16,), i32))
    y = plsc.bitcast(yi, f32)
    for _ in range(3):
        y = y * (1.5 - 0.5 * x * y * y)
    return y


def _fill16(v):
    return jnp.full((16,), v, i32)


def _l1_body(src_hbm, dst_hbm, w_hbm, x_hbm,
             agg_hbm, dinv_hbm, xs_hbm,
             dst_all, w_all, src_all, deg_acc, idx80, dbuf, rows,
             idx_s, idx_d, deg_sh, acc_sh, sem):
    c = lax.axis_index("c")
    s = lax.axis_index("s")
    z16 = jnp.zeros((16,), f32)
    c7 = jnp.full((16,), 7, i32)
    c127 = jnp.full((16,), 127, i32)

    @pl.loop(0, CH)
    def _(r):
        for j in range(8):
            rows[r, pl.ds(j * 16, 16)] = z16

    @pl.loop(0, NPAD // CH)
    def _(r):
        for j in range(8):
            deg_acc[r, pl.ds(j * 16, 16)] = z16

    for m in range(NPAD // CH // 16):
        idx80[pl.ds(m * 16, 16)] = lax.iota(i32, 16) + m * 16

    for m in range(SLICE // CH):
        pltpu.sync_copy(rows, acc_sh.at[pl.ds(s * SLICE + m * CH, CH), :])
    pltpu.sync_copy(rows.at[pl.ds(0, NPAD // CH // NS), :],
                    deg_sh.at[pl.ds(s * (NPAD // CH // NS), NPAD // CH // NS), :])

    plsc.subcore_barrier()

    ks0 = (s * NCH) // NS
    ks1 = ((s + 1) * NCH) // NS
    for t in range(4):
        seg0 = ks0 + t * SEG
        nk = jnp.minimum(SEG, ks1 - seg0)

        @pl.when(nk > 0)
        def _():
            pltpu.sync_copy(dst_hbm.at[pl.ds(seg0 * CH, SEGE)], dst_all)
            pltpu.sync_copy(w_hbm.at[pl.ds(seg0 * CH, SEGE)], w_all)

            @pl.loop(0, nk * (CH // 16))
            def _(g):
                d16 = dst_all[pl.ds(g * 16, 16)]
                w16 = w_all[pl.ds(g * 16, 16)]
                plsc.addupdate_scatter(
                    deg_acc,
                    [lax.shift_right_logical(d16, c7), jnp.bitwise_and(d16, c127)],
                    w16)

    pltpu.sync_copy(deg_acc, deg_sh.at[idx80], add=True)
    plsc.subcore_barrier()

    nrow = NPAD // CH // NS
    pltpu.sync_copy(deg_sh.at[pl.ds(s * nrow, nrow), :], dbuf)
    for r in range(nrow):
        for j in range(8):
            d = dbuf[r, pl.ds(j * 16, 16)]
            dbuf[r, pl.ds(j * 16, 16)] = _rsqrt16(d + 1.0)

    @pl.when(c == 0)
    def _():
        for r in range(nrow):
            pltpu.sync_copy(dbuf.at[r],
                            dinv_hbm.at[pl.ds(s * SLICE + r * CH, CH)])

    for m in range(SLICE // CH):
        pltpu.sync_copy(x_hbm.at[pl.ds(s * SLICE + m * CH, CH), :], rows)

        @pl.loop(0, CH)
        def _(r):
            sp = plsc.load_gather(dbuf, [_fill16(m), _fill16(r)])
            for j in range(8):
                rows[r, pl.ds(j * 16, 16)] = rows[r, pl.ds(j * 16, 16)] * sp
        pltpu.sync_copy(rows, xs_hbm.at[pl.ds(s * SLICE + m * CH, CH), :])

    plsc.subcore_barrier()

    ka0 = c * (NCH // NC) + (s * (NCH // NC)) // NS
    ka1 = c * (NCH // NC) + ((s + 1) * (NCH // NC)) // NS
    for t in range(2):
        seg0 = ka0 + t * SEG
        nk = jnp.minimum(SEG, ka1 - seg0)

        @pl.when(nk > 0)
        def _():
            pltpu.sync_copy(src_hbm.at[pl.ds(seg0 * CH, SEGE)], src_all)
            pltpu.sync_copy(dst_hbm.at[pl.ds(seg0 * CH, SEGE)], dst_all)
            pltpu.sync_copy(w_hbm.at[pl.ds(seg0 * CH, SEGE)], w_all)

            @pl.loop(0, nk)
            def _(k):
                base = k * CH
                for g in range(8):
                    idx_s[pl.ds(g * 16, 16)] = src_all[pl.ds(base + g * 16, 16)]
                    idx_d[pl.ds(g * 16, 16)] = dst_all[pl.ds(base + g * 16, 16)]
                pltpu.async_copy(xs_hbm.at[idx_s], rows, sem).wait()

                @pl.loop(0, CH)
                def _(r):
                    sp = plsc.load_gather(w_all, [_fill16(base + r)])
                    for j in range(8):
                        rows[r, pl.ds(j * 16, 16)] = (
                            rows[r, pl.ds(j * 16, 16)] * sp)

                pltpu.sync_copy(rows, acc_sh.at[idx_d], add=True)

    plsc.subcore_barrier()
    pltpu.sync_copy(acc_sh.at[pl.ds(s * SLICE, SLICE), :],
                    agg_hbm.at[c, pl.ds(s * SLICE, SLICE), :])


def _sc_layer1(src, dst, w, x):
    mesh = plsc.VectorSubcoreMesh(core_axis_name="c", subcore_axis_name="s",
                                  num_cores=NC, num_subcores=NS)
    return pl.kernel(
        _l1_body,
        out_type=(jax.ShapeDtypeStruct((NC, NPAD, 128), f32),
                  jax.ShapeDtypeStruct((NPAD,), f32),
                  jax.ShapeDtypeStruct((NPAD, 128), f32)),
        mesh=mesh,
        scratch_types=[
            pltpu.VMEM((SEGE,), i32),
            pltpu.VMEM((SEGE,), f32),
            pltpu.VMEM((SEGE,), i32),
            pltpu.VMEM((NPAD // CH, CH), f32),
            pltpu.VMEM((NPAD // CH,), i32),
            pltpu.VMEM((NPAD // CH // NS, CH), f32),
            pltpu.VMEM((CH, 128), f32),
            pltpu.VMEM((CH,), i32),
            pltpu.VMEM((CH,), i32),
            pltpu.VMEM_SHARED((NPAD // CH, CH), f32),
            pltpu.VMEM_SHARED((NPAD, 128), f32),
            pltpu.SemaphoreType.DMA,
        ],
        compiler_params=pltpu.CompilerParams(needs_layout_passes=False),
        name="gcn_sc_layer1",
    )(src, dst, w, x)


def _l2_body(src_hbm, dst_hbm, w_hbm, hws_hbm, agg_hbm,
             src_all, dst_all, w_all, rows, idx_s, idx_d, acc_sh, sem):
    c = lax.axis_index("c")
    s = lax.axis_index("s")
    z16 = jnp.zeros((16,), f32)

    @pl.loop(0, CH)
    def _(r):
        rows[r, pl.ds(0, 16)] = z16
        rows[r, pl.ds(16, 16)] = z16
    for m in range(SLICE // CH):
        pltpu.sync_copy(rows, acc_sh.at[pl.ds(s * SLICE + m * CH, CH), :])
    plsc.subcore_barrier()

    ka0 = c * (NCH // NC) + (s * (NCH // NC)) // NS
    ka1 = c * (NCH // NC) + ((s + 1) * (NCH // NC)) // NS
    pltpu.sync_copy(src_hbm.at[pl.ds(ka0 * CH, AGG_BUF)], src_all)
    pltpu.sync_copy(dst_hbm.at[pl.ds(ka0 * CH, AGG_BUF)], dst_all)
    pltpu.sync_copy(w_hbm.at[pl.ds(ka0 * CH, AGG_BUF)], w_all)

    @pl.loop(0, ka1 - ka0)
    def _(k):
        base = k * CH
        for g in range(8):
            idx_s[pl.ds(g * 16, 16)] = src_all[pl.ds(base + g * 16, 16)]
            idx_d[pl.ds(g * 16, 16)] = dst_all[pl.ds(base + g * 16, 16)]
        pltpu.async_copy(hws_hbm.at[idx_s], rows, sem).wait()

        @pl.loop(0, CH)
        def _(r):
            sp = plsc.load_gather(w_all, [_fill16(base + r)])
            rows[r, pl.ds(0, 16)] = rows[r, pl.ds(0, 16)] * sp
            rows[r, pl.ds(16, 16)] = rows[r, pl.ds(16, 16)] * sp

        pltpu.sync_copy(rows, acc_sh.at[idx_d], add=True)

    plsc.subcore_barrier()
    pltpu.sync_copy(acc_sh.at[pl.ds(s * SLICE, SLICE), :],
                    agg_hbm.at[c, pl.ds(s * SLICE, SLICE), :])


def _sc_layer2(src, dst, w, hws):
    mesh = plsc.VectorSubcoreMesh(core_axis_name="c", subcore_axis_name="s",
                                  num_cores=NC, num_subcores=NS)
    return pl.kernel(
        _l2_body,
        out_type=jax.ShapeDtypeStruct((NC, NPAD, 32), f32),
        mesh=mesh,
        scratch_types=[
            pltpu.VMEM((AGG_BUF,), i32),
            pltpu.VMEM((AGG_BUF,), i32),
            pltpu.VMEM((AGG_BUF,), f32),
            pltpu.VMEM((CH, 32), f32),
            pltpu.VMEM((CH,), i32),
            pltpu.VMEM((CH,), i32),
            pltpu.VMEM_SHARED((NPAD, 32), f32),
            pltpu.SemaphoreType.DMA,
        ],
        compiler_params=pltpu.CompilerParams(needs_layout_passes=False,
                                             use_tc_tiling_on_sc=False),
        name="gcn_sc_layer2",
    )(src, dst, w, hws)


BM = 200


def _tc_mid_body(a0, a1, xr, dv, w1, b1, w2, hw_out, hws_out):
    d = dv[...]
    z = (a0[...] + a1[...]) * d + xr[...] * (d * d)
    h = jnp.dot(z, w1[...], preferred_element_type=f32) + b1[...]
    h = jnp.maximum(h, 0.0)
    hw = jnp.dot(h, w2[...], preferred_element_type=f32)
    hw_out[...] = hw
    hws_out[...] = hw * d


def _tc_mid(a0, a1, x, dinv_col, W1p, b1p, W2p):
    return pl.pallas_call(
        _tc_mid_body,
        grid=(NNODE // BM,),
        in_specs=[
            pl.BlockSpec((BM, 128), lambda i: (i, 0)),
            pl.BlockSpec((BM, 128), lambda i: (i, 0)),
            pl.BlockSpec((BM, 128), lambda i: (i, 0)),
            pl.BlockSpec((BM, 1), lambda i: (i, 0)),
            pl.BlockSpec((128, 256), lambda i: (0, 0)),
            pl.BlockSpec((1, 256), lambda i: (0, 0)),
            pl.BlockSpec((256, 32), lambda i: (0, 0)),
        ],
        out_specs=[
            pl.BlockSpec((BM, 32), lambda i: (i, 0)),
            pl.BlockSpec((BM, 32), lambda i: (i, 0)),
        ],
        out_shape=[
            jax.ShapeDtypeStruct((NNODE, 32), f32),
            jax.ShapeDtypeStruct((NNODE, 32), f32),
        ],
        name="gcn_tc_mid",
    )(a0, a1, x, dinv_col, W1p, b1p, W2p)


def _tc_fin_body(q0, q1, hwr, dv, b2, out):
    d = dv[...]
    out[...] = (q0[...] + q1[...]) * d + hwr[...] * (d * d) + b2[...]


def _tc_fin(q0, q1, hw, dinv_col, b2p):
    return pl.pallas_call(
        _tc_fin_body,
        grid=(NNODE // BM,),
        in_specs=[
            pl.BlockSpec((BM, 32), lambda i: (i, 0)),
            pl.BlockSpec((BM, 32), lambda i: (i, 0)),
            pl.BlockSpec((BM, 32), lambda i: (i, 0)),
            pl.BlockSpec((BM, 1), lambda i: (i, 0)),
            pl.BlockSpec((1, 32), lambda i: (0, 0)),
        ],
        out_specs=pl.BlockSpec((BM, 32), lambda i: (i, 0)),
        out_shape=jax.ShapeDtypeStruct((NNODE, 32), f32),
        name="gcn_tc_fin",
    )(q0, q1, hw, dinv_col, b2p)


def kernel(x, edge_index, edge_weight, W1, b1, W2, b2):
    src = jnp.pad(edge_index[0], (0, EBUF - NEDGE))
    dst = jnp.pad(edge_index[1], (0, EBUF - NEDGE))
    w = jnp.pad(edge_weight, (0, EBUF - NEDGE))
    xp = jnp.pad(x, ((0, NPAD - NNODE), (0, 0)))
    agg1, dinv1d, _ = _sc_layer1(src, dst, w, xp)
    dinv_col = dinv1d[:NNODE].reshape(NNODE, 1)
    W1p = jnp.pad(W1, ((0, 0), (0, 56)))
    b1p = jnp.pad(b1, (0, 56)).reshape(1, 256)
    W2p = jnp.pad(W2, ((0, 56), (0, 12)))
    hw, hws = _tc_mid(agg1[0, :NNODE], agg1[1, :NNODE], x, dinv_col,
                      W1p, b1p, W2p)
    agg2 = _sc_layer2(src, dst, edge_weight, hws)
    b2p = jnp.pad(b2, (0, 12)).reshape(1, 32)
    out = _tc_fin(agg2[0, :NNODE], agg2[1, :NNODE], hw, dinv_col, b2p)
    return out[:, :20]
